# baseline (speedup 1.0000x reference)
import functools

import jax
import jax.numpy as jnp
from jax import lax
from jax.experimental import pallas as pl
from jax.experimental.pallas import tpu as pltpu
from jax.experimental.pallas import tpu_sc as plsc

DIM = 16
NUM_FEAT = 26
BATCH = 16384
NC, NS, L = 2, 16, 16
NW = NC * NS
B_PER_W = BATCH // NW
G = 64
NQ = B_PER_W // G


def _sc_embed(xT, off2, table):
    mesh = plsc.VectorSubcoreMesh(core_axis_name="c", subcore_axis_name="s")

    @functools.partial(
        pl.kernel,
        mesh=mesh,
        out_type=jax.ShapeDtypeStruct((NUM_FEAT, DIM, BATCH), jnp.float32),
        compiler_params=pltpu.CompilerParams(
            use_tc_tiling_on_sc=False, needs_layout_passes=False),
        scratch_types=[
            pltpu.VMEM((NUM_FEAT, B_PER_W), jnp.int32),
            pltpu.VMEM((NUM_FEAT, L), jnp.int32),
            pltpu.VMEM((2, NUM_FEAT, G, DIM), jnp.float32),
            pltpu.VMEM((2, NUM_FEAT, DIM, G), jnp.float32),
            pltpu.SemaphoreType.DMA,
            pltpu.SemaphoreType.DMA,
            pltpu.SemaphoreType.DMA,
            pltpu.SemaphoreType.DMA,
        ],
    )
    def k(xT_hbm, off2_hbm, table_hbm, out_hbm, idx_v, off_v, rows_v, trows_v,
          gsem0, gsem1, wsem0, wsem1):
        gsems = (gsem0, gsem1)
        wsems = (wsem0, wsem1)
        wid = lax.axis_index("s") * NC + lax.axis_index("c")
        b0 = wid * B_PER_W

        pltpu.sync_copy(xT_hbm.at[:, pl.ds(b0, B_PER_W)], idx_v)
        pltpu.sync_copy(off2_hbm, off_v)

        def add_body(f, carry):
            off = off_v[f, :]
            for c in range(B_PER_W // L):
                sl = pl.ds(c * L, L)
                idx_v[f, sl] = lax.shift_left(idx_v[f, sl] + off, 3)
            return carry

        lax.fori_loop(0, NUM_FEAT, add_body, 0)

        iota = lax.iota(jnp.int32, L)

        def fire_gathers(q):
            b = q % 2
            return [
                pltpu.async_copy(
                    table_hbm.at[idx_v.at[f, pl.ds(q * G, G)]],
                    rows_v.at[b, f],
                    gsems[b],
                )
                for f in range(NUM_FEAT)
            ]

        def transpose_chunk(b):
            def body(f, carry):
                rows_f = rows_v.at[b, f]
                for d in range(DIM):
                    idx1 = jnp.full((L,), d, jnp.int32)
                    for c in range(G // L):
                        v = plsc.load_gather(rows_f, [iota + (c * L), idx1])
                        trows_v[b, f, d, pl.ds(c * L, L)] = v
                return carry
            lax.fori_loop(0, NUM_FEAT, body, 0)

        pending_g = fire_gathers(0)
        pending_w = [None, None]
        for q in range(NQ):
            b = q % 2
            if q + 1 < NQ:
                next_g = fire_gathers(q + 1)
            for h in pending_g:
                h.wait()
            if pending_w[b] is not None:
                pending_w[b].wait()
            transpose_chunk(b)
            if q + 1 < NQ:
                pending_g = next_g
            pending_w[b] = pltpu.async_copy(
                trows_v.at[b],
                out_hbm.at[:, :, pl.ds(b0 + q * G, G)],
                wsems[b],
            )
        pending_w[0].wait()
        pending_w[1].wait()

    return k(xT, off2, table)


def kernel(x_cat, table, offsets):
    xT = x_cat.astype(jnp.int32).T
    off2 = jnp.broadcast_to(offsets.astype(jnp.int32)[:, None], (NUM_FEAT, L))
    table_pad = jnp.pad(table, ((0, 6), (0, 112)))
    table8 = table_pad.reshape((999992 * 8, DIM))
    out3 = _sc_embed(xT, off2, table8)
    return jnp.transpose(out3, (2, 0, 1))
